# baseline (device time: 15773 ns/iter reference)
import jax
import jax.numpy as jnp
from jax import lax
from jax.experimental import pallas as pl
from jax.experimental.pallas import tpu as pltpu

B, H, D = 8, 8, 64
KLOC = 512
NYZ = 8
KSUB = KLOC // NYZ
NDEV = 16
SCALE = D ** -0.5

_POSITIONS = [(qx, qy, qz, (qx * 2 + qy) * 4 + qz)
              for qx in range(2) for qy in range(2) for qz in range(4)]


def kernel(Q, K, V):
    Q2 = Q.reshape(B, H, D)
    K2 = K.reshape(B, KLOC, H * D)
    V2 = V.reshape(B, KLOC, H * D)
    start = (lax.axis_index("y") * 4 + lax.axis_index("z")) * KSUB
    Ks = lax.dynamic_slice_in_dim(K2, start, KSUB, axis=1)
    Vs = lax.dynamic_slice_in_dim(V2, start, KSUB, axis=1)

    def body(q_ref, k_ref, v_ref, o_ref, comm, s_sems, r_sems):
        my_x = lax.axis_index("x")
        my_y = lax.axis_index("y")
        my_z = lax.axis_index("z")
        my_lin = (my_x * 2 + my_y) * 4 + my_z

        rowh = lax.broadcasted_iota(jnp.int32, (H * D, H), 0) // D
        colh = lax.broadcasted_iota(jnp.int32, (H * D, H), 1)
        qmaskT = (rowh == colh).astype(jnp.float32)
        eye3 = (lax.broadcasted_iota(jnp.int32, (H, H, 1), 0)
                == lax.broadcasted_iota(jnp.int32, (H, H, 1), 1)
                ).astype(jnp.float32)

        ms, ls, os_ = [], [], []
        for b in range(B):
            qbT = q_ref[b].T
            qblkT = jnp.concatenate([qbT] * H, axis=0) * qmaskT
            s = lax.dot_general(
                k_ref[b], qblkT, (((1,), (0,)), ((), ())),
                preferred_element_type=jnp.float32) * SCALE
            m = jnp.max(s, axis=0, keepdims=True)
            p = jnp.exp(s - m)
            l = jnp.sum(p, axis=0, keepdims=True)
            t = lax.dot_general(
                p, v_ref[b], (((0,), (0,)), ((), ())),
                preferred_element_type=jnp.float32)
            ob = jnp.sum(t.reshape(H, H, D) * eye3, axis=0)
            ms.append(m)
            ls.append(l)
            os_.append(ob)

        m_arr = jnp.concatenate(ms, axis=0)
        l_arr = jnp.concatenate(ls, axis=0)
        stat = jnp.concatenate(
            [m_arr, l_arr, jnp.zeros((B, D - 2 * H), jnp.float32)], axis=1)
        msg = jnp.concatenate(
            [jnp.stack(os_, axis=0), stat[None]], axis=0)
        comm[my_lin] = msg.astype(jnp.bfloat16)

        bar = pltpu.get_barrier_semaphore()
        for qx, qy, qz, lin_q in _POSITIONS:
            @pl.when(lin_q != my_lin)
            def _(qx=qx, qy=qy, qz=qz):
                pl.semaphore_signal(bar, inc=1, device_id=(qx, qy, qz),
                                    device_id_type=pl.DeviceIdType.MESH)
        pl.semaphore_wait(bar, NDEV - 1)

        def out_desc(qx, qy, qz, lin_q):
            return pltpu.make_async_remote_copy(
                src_ref=comm.at[my_lin], dst_ref=comm.at[my_lin],
                send_sem=s_sems.at[lin_q], recv_sem=r_sems.at[my_lin],
                device_id=(qx, qy, qz), device_id_type=pl.DeviceIdType.MESH)

        for qx, qy, qz, lin_q in _POSITIONS:
            @pl.when(lin_q != my_lin)
            def _(qx=qx, qy=qy, qz=qz, lin_q=lin_q):
                out_desc(qx, qy, qz, lin_q).start()

        for qx, qy, qz, lin_q in _POSITIONS:
            @pl.when(lin_q != my_lin)
            def _(qx=qx, qy=qy, qz=qz, lin_q=lin_q):
                pltpu.make_async_remote_copy(
                    src_ref=comm.at[lin_q], dst_ref=comm.at[lin_q],
                    send_sem=s_sems.at[lin_q], recv_sem=r_sems.at[lin_q],
                    device_id=(qx, qy, qz),
                    device_id_type=pl.DeviceIdType.MESH).wait_recv()

        call = comm[...].astype(jnp.float32)
        o_all = call[:, :B]
        m_all = call[:, B, :, 0:H]
        l_all = call[:, B, :, H:2 * H]
        m_n = jnp.max(m_all, axis=0)
        w = jnp.exp(m_all - m_n[None])
        l_n = jnp.sum(w * l_all, axis=0)
        o = jnp.sum(w[..., None] * o_all, axis=0) / l_n[..., None]
        o_ref[...] = o[:, None]

        for qx, qy, qz, lin_q in _POSITIONS:
            @pl.when(lin_q != my_lin)
            def _(qx=qx, qy=qy, qz=qz, lin_q=lin_q):
                out_desc(qx, qy, qz, lin_q).wait_send()

    return pl.pallas_call(
        body,
        out_shape=jax.ShapeDtypeStruct((B, 1, H, D), jnp.float32),
        in_specs=[
            pl.BlockSpec(memory_space=pltpu.VMEM),
            pl.BlockSpec(memory_space=pltpu.VMEM),
            pl.BlockSpec(memory_space=pltpu.VMEM),
        ],
        out_specs=pl.BlockSpec(memory_space=pltpu.VMEM),
        scratch_shapes=[
            pltpu.VMEM((NDEV, B + 1, H, D), jnp.bfloat16),
            pltpu.SemaphoreType.DMA((NDEV,)),
            pltpu.SemaphoreType.DMA((NDEV,)),
        ],
        compiler_params=pltpu.CompilerParams(collective_id=0),
    )(Q2, Ks, Vs)
